# baseline (speedup 1.0000x reference)
import jax
import jax.numpy as jnp
from jax.experimental import pallas as pl
from jax.experimental.pallas import tpu as pltpu

N = 1024
IN_F = 128
OUT_F = 64
LOG2E = 1.4426950408889634
CHUNK = 512
NCH = N // CHUNK


def _gat_kernel(x_ref, adj_hbm, w_ref, a_ref, out_ref, abuf, sem):
    def chunk_copy(c, buf_slot):
        return pltpu.make_async_copy(
            adj_hbm.at[pl.ds(c * CHUNK, CHUNK), :], abuf.at[buf_slot],
            sem.at[buf_slot])

    chunk_copy(0, 0).start()

    h = jnp.dot(x_ref[...], w_ref[...], preferred_element_type=jnp.float32)
    a_vec = a_ref[...]
    f = jnp.dot(h, a_vec[:OUT_F, :], preferred_element_type=jnp.float32)
    g = jnp.dot(h, a_vec[OUT_F:, :], preferred_element_type=jnp.float32)
    fg = f + jnp.max(g)
    mhat = jnp.maximum(fg, 0.2 * fg)
    u = (f - mhat) * LOG2E
    v = (0.2 * f - mhat) * LOG2E
    g_row = g.reshape(1, N) * LOG2E
    g2_row = 0.2 * g_row
    ones = jnp.ones((N, 1), dtype=jnp.float32)
    h_ext = jnp.concatenate([h, ones], axis=1)
    hmean = jnp.sum(h, axis=0, keepdims=True) * (1.0 / N)

    for c in range(NCH):
        if c + 1 < NCH:
            chunk_copy(c + 1, (c + 1) % 2).start()
        chunk_copy(c, c % 2).wait()
        adj_c = abuf[c % 2]
        lo, hi = c * CHUNK, (c + 1) * CHUNK
        e2 = jnp.maximum(u[lo:hi, :] + g_row, v[lo:hi, :] + g2_row)
        p = adj_c * jnp.exp2(e2)
        o_ext = jnp.dot(p, h_ext, preferred_element_type=jnp.float32)
        denom = o_ext[:, OUT_F:]
        o = o_ext[:, :OUT_F] / denom
        o = jnp.where(denom > 0, o, hmean)
        out_ref[lo:hi, :] = jnp.where(o > 0, o, jnp.exp(o) - 1.0)


@jax.jit
def kernel(x, adj, W, a):
    return pl.pallas_call(
        _gat_kernel,
        in_specs=[
            pl.BlockSpec(memory_space=pltpu.VMEM),
            pl.BlockSpec(memory_space=pl.ANY),
            pl.BlockSpec(memory_space=pltpu.VMEM),
            pl.BlockSpec(memory_space=pltpu.VMEM),
        ],
        out_specs=pl.BlockSpec(memory_space=pltpu.VMEM),
        scratch_shapes=[
            pltpu.VMEM((2, CHUNK, N), jnp.float32),
            pltpu.SemaphoreType.DMA((2,)),
        ],
        out_shape=jax.ShapeDtypeStruct((N, OUT_F), jnp.float32),
    )(x, adj, W, a)


# speedup vs baseline: 1.0260x; 1.0260x over previous
import jax
import jax.numpy as jnp
from jax.experimental import pallas as pl
from jax.experimental.pallas import tpu as pltpu

N = 1024
IN_F = 128
OUT_F = 64
LOG2E = 1.4426950408889634
TILE = 64
NT = N // TILE


def _gat_kernel(x_ref, adj_ref, w_ref, a_ref, out_ref, p_ref):
    h = jnp.dot(x_ref[...], w_ref[...], preferred_element_type=jnp.float32)
    a_vec = a_ref[...]
    f = jnp.dot(h, a_vec[:OUT_F, :], preferred_element_type=jnp.float32)
    g = jnp.dot(h, a_vec[OUT_F:, :], preferred_element_type=jnp.float32)
    fg = f + jnp.max(g)
    mhat = jnp.maximum(fg, 0.2 * fg)
    u = (f - mhat) * LOG2E
    v = (0.2 * f - mhat) * LOG2E
    g_row = g.reshape(1, N) * LOG2E
    g2_row = 0.2 * g_row

    for t in range(NT):
        lo, hi = t * TILE, (t + 1) * TILE
        e2 = jnp.maximum(u[lo:hi, :] + g_row, v[lo:hi, :] + g2_row)
        p_ref[lo:hi, :] = adj_ref[lo:hi, :] * jnp.exp2(e2)

    ones = jnp.ones((N, 1), dtype=jnp.float32)
    h_ext = jnp.concatenate([h, ones], axis=1)
    o_ext = jnp.dot(p_ref[...], h_ext, preferred_element_type=jnp.float32)
    denom = o_ext[:, OUT_F:]
    o = o_ext[:, :OUT_F] / denom
    hmean = jnp.sum(h, axis=0, keepdims=True) * (1.0 / N)
    o = jnp.where(denom > 0, o, hmean)
    out_ref[...] = jnp.where(o > 0, o, jnp.exp(o) - 1.0)


@jax.jit
def kernel(x, adj, W, a):
    return pl.pallas_call(
        _gat_kernel,
        scratch_shapes=[pltpu.VMEM((N, N), jnp.float32)],
        out_shape=jax.ShapeDtypeStruct((N, OUT_F), jnp.float32),
    )(x, adj, W, a)
